# baseline (speedup 1.0000x reference)
import functools

import jax
import jax.numpy as jnp
from jax import lax
from jax.experimental import pallas as pl
from jax.experimental.pallas import tpu as pltpu
from jax.experimental.pallas import tpu_sc as plsc

LO, HI = 0.8, 5.2
LANES = 16


def _build_sc_kernel(batch, emb):
    info = plsc.get_sparse_core_info()
    nw = info.num_cores * info.num_subcores
    nc = info.num_cores
    b_per_w = batch // nw
    chunks = b_per_w // LANES
    mesh = plsc.VectorSubcoreMesh(core_axis_name="c", subcore_axis_name="s")

    @functools.partial(
        pl.kernel,
        out_type=jax.ShapeDtypeStruct((batch,), jnp.float32),
        mesh=mesh,
        scratch_types=[
            pltpu.VMEM((b_per_w,), jnp.int32),
            pltpu.VMEM((b_per_w,), jnp.int32),
            pltpu.VMEM((emb * b_per_w,), jnp.float32),
            pltpu.VMEM((emb * b_per_w,), jnp.float32),
            pltpu.VMEM((b_per_w,), jnp.float32),
            pltpu.VMEM((b_per_w,), jnp.float32),
            pltpu.VMEM((b_per_w,), jnp.float32),
            pltpu.SemaphoreType.DMA,
            pltpu.SemaphoreType.DMA,
        ],
        compiler_params=pltpu.CompilerParams(
            needs_layout_passes=False, use_tc_tiling_on_sc=False),
    )
    def sc_kernel(uidx_hbm, midx_hbm, uembt_hbm, iembt_hbm, ubias_hbm,
                  ibias_hbm, out_hbm,
                  uidx_v, midx_v, ucols_v, icols_v, ubias_v, ibias_v, out_v,
                  sem_emb, sem_bias):
        wid = lax.axis_index("s") * nc + lax.axis_index("c")
        base = wid * b_per_w
        pltpu.sync_copy(uidx_hbm.at[pl.ds(base, b_per_w)], uidx_v)
        pltpu.sync_copy(midx_hbm.at[pl.ds(base, b_per_w)], midx_v)
        cub = pltpu.async_copy(ubias_hbm.at[0].at[uidx_v], ubias_v, sem_bias)
        cib = pltpu.async_copy(ibias_hbm.at[0].at[midx_v], ibias_v, sem_bias)
        copies = []
        for e in range(emb):
            copies.append(pltpu.async_copy(
                uembt_hbm.at[e].at[uidx_v],
                ucols_v.at[pl.ds(e * b_per_w, b_per_w)], sem_emb))
            copies.append(pltpu.async_copy(
                iembt_hbm.at[e].at[midx_v],
                icols_v.at[pl.ds(e * b_per_w, b_per_w)], sem_emb))
        cub.wait()
        cib.wait()
        for c in copies:
            c.wait()

        def chunk_body(c, carry):
            off = c * LANES
            acc = ubias_v[pl.ds(off, LANES)] + ibias_v[pl.ds(off, LANES)]
            for e in range(emb):
                uu = ucols_v[pl.ds(e * b_per_w + off, LANES)]
                vv = icols_v[pl.ds(e * b_per_w + off, LANES)]
                acc = acc + uu * vv
            sig = 1.0 / (1.0 + jnp.exp(-acc))
            out_v[pl.ds(off, LANES)] = sig * (HI - LO) + LO
            return carry

        lax.fori_loop(0, chunks, chunk_body, 0)
        pltpu.sync_copy(out_v, out_hbm.at[pl.ds(base, b_per_w)])

    return sc_kernel


def kernel(user_idx, movie_idx, user_emb_table, item_emb_table,
           user_bias_table, item_bias_table):
    batch = user_idx.shape[0]
    emb = user_emb_table.shape[1]
    sc = _build_sc_kernel(batch, emb)
    return sc(
        user_idx.astype(jnp.int32),
        movie_idx.astype(jnp.int32),
        user_emb_table.T,
        item_emb_table.T,
        user_bias_table.T,
        item_bias_table.T,
    )


# speedup vs baseline: 17.3199x; 17.3199x over previous
import functools

import jax
import jax.numpy as jnp
from jax import lax
from jax.experimental import pallas as pl
from jax.experimental.pallas import tpu as pltpu
from jax.experimental.pallas import tpu_sc as plsc

LO, HI = 0.8, 5.2
LANES = 16
TILE_MINOR = 128
TILE_MAJOR = 8


def _build_sc_kernel(batch, emb, n_main, n_rows):
    info = plsc.get_sparse_core_info()
    nw = info.num_cores * info.num_subcores
    nc = info.num_cores
    b_per_w = batch // nw
    chunks = b_per_w // LANES
    n_tail = n_rows - n_main
    tq = n_main // TILE_MINOR
    te = emb // TILE_MAJOR
    plane = tq * 1024
    mesh = plsc.VectorSubcoreMesh(core_axis_name="c", subcore_axis_name="s")

    @functools.partial(
        pl.kernel,
        out_type=jax.ShapeDtypeStruct((batch,), jnp.float32),
        mesh=mesh,
        scratch_types=[
            pltpu.VMEM((b_per_w,), jnp.int32),
            pltpu.VMEM((b_per_w,), jnp.int32),
            pltpu.VMEM((b_per_w,), jnp.int32),
            pltpu.VMEM((b_per_w,), jnp.int32),
            pltpu.VMEM((b_per_w,), jnp.int32),
            pltpu.VMEM((b_per_w,), jnp.int32),
            pltpu.VMEM((emb * b_per_w,), jnp.float32),
            pltpu.VMEM((emb * b_per_w,), jnp.float32),
            pltpu.VMEM((b_per_w,), jnp.float32),
            pltpu.VMEM((b_per_w,), jnp.float32),
            pltpu.VMEM((emb, n_tail), jnp.float32),
            pltpu.VMEM((emb, n_tail), jnp.float32),
            pltpu.VMEM((n_tail,), jnp.float32),
            pltpu.VMEM((n_tail,), jnp.float32),
            pltpu.VMEM((b_per_w,), jnp.float32),
            pltpu.SemaphoreType.DMA,
            pltpu.SemaphoreType.DMA,
        ],
        compiler_params=pltpu.CompilerParams(
            needs_layout_passes=False, use_tc_tiling_on_sc=False),
    )
    def sc_kernel(uidx_hbm, midx_hbm, uflat_hbm, iflat_hbm, ub_hbm, ib_hbm,
                  utail_hbm, itail_hbm, ubt_hbm, ibt_hbm, out_hbm,
                  uidx_v, midx_v, ubase_v, mbase_v, uclamp_v, mclamp_v,
                  ucols_v, icols_v, ubias_v, ibias_v,
                  utail_v, itail_v, ubt_v, ibt_v, out_v,
                  sem_emb, sem_small):
        wid = lax.axis_index("s") * nc + lax.axis_index("c")
        base = wid * b_per_w
        pltpu.sync_copy(uidx_hbm.at[pl.ds(base, b_per_w)], uidx_v)
        pltpu.sync_copy(midx_hbm.at[pl.ds(base, b_per_w)], midx_v)
        ct = [
            pltpu.async_copy(utail_hbm, utail_v, sem_small),
            pltpu.async_copy(itail_hbm, itail_v, sem_small),
            pltpu.async_copy(ubt_hbm, ubt_v, sem_small),
            pltpu.async_copy(ibt_hbm, ibt_v, sem_small),
        ]

        def addr_body(c, carry):
            off = c * LANES
            for idx_v, base_v, clamp_v in (
                    (uidx_v, ubase_v, uclamp_v), (midx_v, mbase_v, mclamp_v)):
                r = idx_v[pl.ds(off, LANES)]
                rc = jnp.minimum(r, n_main - 1)
                q = lax.shift_right_logical(rc, 7)
                addr = lax.shift_left(q, 10) + (rc - lax.shift_left(q, 7))
                base_v[pl.ds(off, LANES)] = addr
                clamp_v[pl.ds(off, LANES)] = rc
            return carry

        lax.fori_loop(0, chunks, addr_body, 0)

        cub = pltpu.async_copy(ub_hbm.at[uclamp_v], ubias_v, sem_small)
        cib = pltpu.async_copy(ib_hbm.at[mclamp_v], ibias_v, sem_small)
        copies = []
        for e in range(emb):
            off_e = (e // TILE_MAJOR) * plane + (e % TILE_MAJOR) * TILE_MINOR
            need = (tq - 1) * 1024 + TILE_MINOR
            copies.append(pltpu.async_copy(
                uflat_hbm.at[pl.ds(off_e, need)].at[ubase_v],
                ucols_v.at[pl.ds(e * b_per_w, b_per_w)], sem_emb))
            copies.append(pltpu.async_copy(
                iflat_hbm.at[pl.ds(off_e, need)].at[mbase_v],
                icols_v.at[pl.ds(e * b_per_w, b_per_w)], sem_emb))
        for c in ct:
            c.wait()
        cub.wait()
        cib.wait()
        for c in copies:
            c.wait()

        def chunk_body(c, carry):
            off = c * LANES
            ru = uidx_v[pl.ds(off, LANES)]
            ri = midx_v[pl.ds(off, LANES)]
            umask = ru >= n_main
            imask = ri >= n_main
            ub = ubias_v[pl.ds(off, LANES)]
            ib = ibias_v[pl.ds(off, LANES)]
            any_tail = jnp.any(umask | imask)

            ut = jnp.minimum(jnp.maximum(ru - n_main, 0), n_tail - 1)
            it = jnp.minimum(jnp.maximum(ri - n_main, 0), n_tail - 1)

            def dot(patch):
                accs = jnp.zeros((LANES,), jnp.float32)
                ubx, ibx = ub, ib
                if patch:
                    ubx = jnp.where(umask, plsc.load_gather(ubt_v, [ut]), ubx)
                    ibx = jnp.where(imask, plsc.load_gather(ibt_v, [it]), ibx)
                accs = ubx + ibx
                for e in range(emb):
                    uu = ucols_v[pl.ds(e * b_per_w + off, LANES)]
                    vv = icols_v[pl.ds(e * b_per_w + off, LANES)]
                    if patch:
                        ecol = jnp.full((LANES,), e, jnp.int32)
                        uu = jnp.where(
                            umask, plsc.load_gather(utail_v, [ecol, ut]), uu)
                        vv = jnp.where(
                            imask, plsc.load_gather(itail_v, [ecol, it]), vv)
                    accs = accs + uu * vv
                sig = 1.0 / (1.0 + jnp.exp(-accs))
                out_v[pl.ds(off, LANES)] = sig * (HI - LO) + LO

            @pl.when(any_tail)
            def _():
                dot(True)

            @pl.when(jnp.logical_not(any_tail))
            def _():
                dot(False)

            return carry

        lax.fori_loop(0, chunks, chunk_body, 0)
        pltpu.sync_copy(out_v, out_hbm.at[pl.ds(base, b_per_w)])

    return sc_kernel


def kernel(user_idx, movie_idx, user_emb_table, item_emb_table,
           user_bias_table, item_bias_table):
    batch = user_idx.shape[0]
    n_rows, emb = user_emb_table.shape
    n_main = (n_rows // TILE_MINOR) * TILE_MINOR
    tq = n_main // TILE_MINOR
    te = emb // TILE_MAJOR

    def flat_view(t):
        return (t[:n_main].T
                .reshape(te, TILE_MAJOR, tq, TILE_MINOR)
                .transpose(0, 2, 1, 3)
                .reshape(-1))

    sc = _build_sc_kernel(batch, emb, n_main, n_rows)
    return sc(
        user_idx.astype(jnp.int32),
        movie_idx.astype(jnp.int32),
        flat_view(user_emb_table),
        flat_view(item_emb_table),
        user_bias_table[:n_main].reshape(-1),
        item_bias_table[:n_main].reshape(-1),
        user_emb_table[n_main:].T,
        item_emb_table[n_main:].T,
        user_bias_table[n_main:].reshape(-1),
        item_bias_table[n_main:].reshape(-1),
    )


# speedup vs baseline: 17.3367x; 1.0010x over previous
import functools

import jax
import jax.numpy as jnp
from jax import lax
from jax.experimental import pallas as pl
from jax.experimental.pallas import tpu as pltpu
from jax.experimental.pallas import tpu_sc as plsc

LO, HI = 0.8, 5.2
LANES = 16
TILE_MINOR = 128
TILE_MAJOR = 8


def _build_sc_kernel(batch, emb, n_main, n_rows):
    info = plsc.get_sparse_core_info()
    nw = info.num_cores * info.num_subcores
    nc = info.num_cores
    b_per_w = batch // nw
    chunks = b_per_w // LANES
    n_tail = n_rows - n_main
    tq = n_main // TILE_MINOR
    te = emb // TILE_MAJOR
    plane = tq * 1024
    mesh = plsc.VectorSubcoreMesh(core_axis_name="c", subcore_axis_name="s")

    @functools.partial(
        pl.kernel,
        out_type=jax.ShapeDtypeStruct((batch,), jnp.float32),
        mesh=mesh,
        scratch_types=[
            pltpu.VMEM((b_per_w,), jnp.int32),
            pltpu.VMEM((b_per_w,), jnp.int32),
            pltpu.VMEM((b_per_w,), jnp.int32),
            pltpu.VMEM((b_per_w,), jnp.int32),
            pltpu.VMEM((b_per_w,), jnp.int32),
            pltpu.VMEM((b_per_w,), jnp.int32),
            pltpu.VMEM((emb * b_per_w,), jnp.float32),
            pltpu.VMEM((emb * b_per_w,), jnp.float32),
            pltpu.VMEM((b_per_w,), jnp.float32),
            pltpu.VMEM((b_per_w,), jnp.float32),
            pltpu.VMEM((emb, n_tail), jnp.float32),
            pltpu.VMEM((emb, n_tail), jnp.float32),
            pltpu.VMEM((n_tail,), jnp.float32),
            pltpu.VMEM((n_tail,), jnp.float32),
            pltpu.VMEM((b_per_w,), jnp.float32),
            pltpu.SemaphoreType.DMA,
            pltpu.SemaphoreType.DMA,
        ],
        compiler_params=pltpu.CompilerParams(
            needs_layout_passes=False, use_tc_tiling_on_sc=False),
    )
    def sc_kernel(uidx_hbm, midx_hbm, uflat_hbm, iflat_hbm, ub_hbm, ib_hbm,
                  utail_hbm, itail_hbm, ubt_hbm, ibt_hbm, out_hbm,
                  uidx_v, midx_v, ubase_v, mbase_v, uclamp_v, mclamp_v,
                  ucols_v, icols_v, ubias_v, ibias_v,
                  utail_v, itail_v, ubt_v, ibt_v, out_v,
                  sem_emb, sem_small):
        wid = lax.axis_index("s") * nc + lax.axis_index("c")
        base = wid * b_per_w
        pltpu.sync_copy(uidx_hbm.at[pl.ds(base, b_per_w)], uidx_v)
        pltpu.sync_copy(midx_hbm.at[pl.ds(base, b_per_w)], midx_v)
        ct = [
            pltpu.async_copy(utail_hbm, utail_v, sem_small),
            pltpu.async_copy(itail_hbm, itail_v, sem_small),
            pltpu.async_copy(ubt_hbm, ubt_v, sem_small),
            pltpu.async_copy(ibt_hbm, ibt_v, sem_small),
        ]

        def addr_body(c, carry):
            off = c * LANES
            for idx_v, base_v, clamp_v in (
                    (uidx_v, ubase_v, uclamp_v), (midx_v, mbase_v, mclamp_v)):
                r = idx_v[pl.ds(off, LANES)]
                rc = jnp.minimum(r, n_main - 1)
                q = lax.shift_right_logical(rc, 7)
                addr = lax.shift_left(q, 10) + (rc - lax.shift_left(q, 7))
                base_v[pl.ds(off, LANES)] = addr
                clamp_v[pl.ds(off, LANES)] = rc
            return carry

        lax.fori_loop(0, chunks, addr_body, 0)

        cub = pltpu.async_copy(ub_hbm.at[0].at[uclamp_v], ubias_v, sem_small)
        cib = pltpu.async_copy(ib_hbm.at[0].at[mclamp_v], ibias_v, sem_small)
        copies = []
        for e in range(emb):
            off_e = (e // TILE_MAJOR) * plane + (e % TILE_MAJOR) * TILE_MINOR
            need = (tq - 1) * 1024 + TILE_MINOR
            copies.append(pltpu.async_copy(
                uflat_hbm.at[pl.ds(off_e, need)].at[ubase_v],
                ucols_v.at[pl.ds(e * b_per_w, b_per_w)], sem_emb))
            copies.append(pltpu.async_copy(
                iflat_hbm.at[pl.ds(off_e, need)].at[mbase_v],
                icols_v.at[pl.ds(e * b_per_w, b_per_w)], sem_emb))
        for c in ct:
            c.wait()
        cub.wait()
        cib.wait()
        for c in copies:
            c.wait()

        def chunk_body(c, carry):
            off = c * LANES
            ru = uidx_v[pl.ds(off, LANES)]
            ri = midx_v[pl.ds(off, LANES)]
            umask = ru >= n_main
            imask = ri >= n_main
            ub = ubias_v[pl.ds(off, LANES)]
            ib = ibias_v[pl.ds(off, LANES)]
            any_tail = jnp.any(umask | imask)

            ut = jnp.minimum(jnp.maximum(ru - n_main, 0), n_tail - 1)
            it = jnp.minimum(jnp.maximum(ri - n_main, 0), n_tail - 1)

            def dot(patch):
                accs = jnp.zeros((LANES,), jnp.float32)
                ubx, ibx = ub, ib
                if patch:
                    ubx = jnp.where(umask, plsc.load_gather(ubt_v, [ut]), ubx)
                    ibx = jnp.where(imask, plsc.load_gather(ibt_v, [it]), ibx)
                accs = ubx + ibx
                for e in range(emb):
                    uu = ucols_v[pl.ds(e * b_per_w + off, LANES)]
                    vv = icols_v[pl.ds(e * b_per_w + off, LANES)]
                    if patch:
                        ecol = jnp.full((LANES,), e, jnp.int32)
                        uu = jnp.where(
                            umask, plsc.load_gather(utail_v, [ecol, ut]), uu)
                        vv = jnp.where(
                            imask, plsc.load_gather(itail_v, [ecol, it]), vv)
                    accs = accs + uu * vv
                sig = 1.0 / (1.0 + jnp.exp(-accs))
                out_v[pl.ds(off, LANES)] = sig * (HI - LO) + LO

            @pl.when(any_tail)
            def _():
                dot(True)

            @pl.when(jnp.logical_not(any_tail))
            def _():
                dot(False)

            return carry

        lax.fori_loop(0, chunks, chunk_body, 0)
        pltpu.sync_copy(out_v, out_hbm.at[pl.ds(base, b_per_w)])

    return sc_kernel


def kernel(user_idx, movie_idx, user_emb_table, item_emb_table,
           user_bias_table, item_bias_table):
    batch = user_idx.shape[0]
    n_rows, emb = user_emb_table.shape
    n_main = (n_rows // TILE_MINOR) * TILE_MINOR
    tq = n_main // TILE_MINOR
    te = emb // TILE_MAJOR

    def flat_view(t):
        return (t[:n_main].T
                .reshape(te, TILE_MAJOR, tq, TILE_MINOR)
                .transpose(0, 2, 1, 3)
                .reshape(-1))

    sc = _build_sc_kernel(batch, emb, n_main, n_rows)
    return sc(
        user_idx.astype(jnp.int32),
        movie_idx.astype(jnp.int32),
        flat_view(user_emb_table),
        flat_view(item_emb_table),
        user_bias_table.T[:, :n_main],
        item_bias_table.T[:, :n_main],
        user_emb_table[n_main:].T,
        item_emb_table[n_main:].T,
        user_bias_table[n_main:].reshape(-1),
        item_bias_table[n_main:].reshape(-1),
    )


# speedup vs baseline: 17.7142x; 1.0218x over previous
import functools

import jax
import jax.numpy as jnp
from jax import lax
from jax.experimental import pallas as pl
from jax.experimental.pallas import tpu as pltpu
from jax.experimental.pallas import tpu_sc as plsc

LO, HI = 0.8, 5.2
LANES = 16
TILE_MINOR = 128
TILE_MAJOR = 8


def _build_sc_kernel(batch, emb, n_main, n_rows):
    info = plsc.get_sparse_core_info()
    nw = info.num_cores * info.num_subcores
    nc = info.num_cores
    b_per_w = batch // nw
    chunks = b_per_w // LANES
    n_tail = n_rows - n_main
    tq = n_main // TILE_MINOR
    te = emb // TILE_MAJOR
    plane = tq * 1024
    mesh = plsc.VectorSubcoreMesh(core_axis_name="c", subcore_axis_name="s")

    @functools.partial(
        pl.kernel,
        out_type=jax.ShapeDtypeStruct((batch,), jnp.float32),
        mesh=mesh,
        scratch_types=[
            pltpu.VMEM((b_per_w,), jnp.int32),
            pltpu.VMEM((b_per_w,), jnp.int32),
            pltpu.VMEM((b_per_w,), jnp.int32),
            pltpu.VMEM((b_per_w,), jnp.int32),
            pltpu.VMEM((b_per_w,), jnp.int32),
            pltpu.VMEM((b_per_w,), jnp.int32),
            pltpu.VMEM((emb * b_per_w,), jnp.float32),
            pltpu.VMEM((emb * b_per_w,), jnp.float32),
            pltpu.VMEM((b_per_w,), jnp.float32),
            pltpu.VMEM((b_per_w,), jnp.float32),
            pltpu.VMEM((emb, n_tail), jnp.float32),
            pltpu.VMEM((emb, n_tail), jnp.float32),
            pltpu.VMEM((n_tail,), jnp.float32),
            pltpu.VMEM((n_tail,), jnp.float32),
            pltpu.VMEM((b_per_w,), jnp.float32),
            pltpu.SemaphoreType.DMA,
            pltpu.SemaphoreType.DMA,
        ],
        compiler_params=pltpu.CompilerParams(
            needs_layout_passes=False, use_tc_tiling_on_sc=False),
    )
    def sc_kernel(uidx_hbm, midx_hbm, uflat_hbm, iflat_hbm, ub_hbm, ib_hbm,
                  utail_hbm, itail_hbm, ubt_hbm, ibt_hbm, out_hbm,
                  uidx_v, midx_v, ubase_v, mbase_v, uclamp_v, mclamp_v,
                  ucols_v, icols_v, ubias_v, ibias_v,
                  utail_v, itail_v, ubt_v, ibt_v, out_v,
                  sem_emb, sem_small):
        wid = lax.axis_index("s") * nc + lax.axis_index("c")
        base = wid * b_per_w
        pltpu.sync_copy(uidx_hbm.at[pl.ds(base, b_per_w)], uidx_v)
        pltpu.sync_copy(midx_hbm.at[pl.ds(base, b_per_w)], midx_v)
        ct = [
            pltpu.async_copy(utail_hbm, utail_v, sem_small),
            pltpu.async_copy(itail_hbm, itail_v, sem_small),
            pltpu.async_copy(ubt_hbm, ubt_v, sem_small),
            pltpu.async_copy(ibt_hbm, ibt_v, sem_small),
        ]

        def addr_body(c, carry):
            off = c * LANES
            for idx_v, base_v, clamp_v in (
                    (uidx_v, ubase_v, uclamp_v), (midx_v, mbase_v, mclamp_v)):
                r = idx_v[pl.ds(off, LANES)]
                rc = jnp.minimum(r, n_main - 1)
                q = lax.shift_right_logical(rc, 7)
                addr = lax.shift_left(q, 10) + (rc - lax.shift_left(q, 7))
                base_v[pl.ds(off, LANES)] = addr
                clamp_v[pl.ds(off, LANES)] = rc
            return carry

        lax.fori_loop(0, chunks, addr_body, 0)

        cub = pltpu.async_copy(ub_hbm.at[0].at[uidx_v], ubias_v, sem_small)
        cib = pltpu.async_copy(ib_hbm.at[0].at[midx_v], ibias_v, sem_small)
        copies = []
        for e in range(emb):
            off_e = (e // TILE_MAJOR) * plane + (e % TILE_MAJOR) * TILE_MINOR
            need = (tq - 1) * 1024 + TILE_MINOR
            copies.append(pltpu.async_copy(
                uflat_hbm.at[pl.ds(off_e, need)].at[ubase_v],
                ucols_v.at[pl.ds(e * b_per_w, b_per_w)], sem_emb))
            copies.append(pltpu.async_copy(
                iflat_hbm.at[pl.ds(off_e, need)].at[mbase_v],
                icols_v.at[pl.ds(e * b_per_w, b_per_w)], sem_emb))
        for c in ct:
            c.wait()
        cub.wait()
        cib.wait()
        for c in copies:
            c.wait()

        def chunk_body(c, carry):
            off = c * LANES
            ru = uidx_v[pl.ds(off, LANES)]
            ri = midx_v[pl.ds(off, LANES)]
            umask = ru >= n_main
            imask = ri >= n_main
            ub = ubias_v[pl.ds(off, LANES)]
            ib = ibias_v[pl.ds(off, LANES)]
            any_tail = jnp.any(umask | imask)

            ut = jnp.minimum(jnp.maximum(ru - n_main, 0), n_tail - 1)
            it = jnp.minimum(jnp.maximum(ri - n_main, 0), n_tail - 1)

            def dot(patch):
                accs = jnp.zeros((LANES,), jnp.float32)
                accs = ub + ib
                for e in range(emb):
                    uu = ucols_v[pl.ds(e * b_per_w + off, LANES)]
                    vv = icols_v[pl.ds(e * b_per_w + off, LANES)]
                    if patch:
                        ecol = jnp.full((LANES,), e, jnp.int32)
                        uu = jnp.where(
                            umask, plsc.load_gather(utail_v, [ecol, ut]), uu)
                        vv = jnp.where(
                            imask, plsc.load_gather(itail_v, [ecol, it]), vv)
                    accs = accs + uu * vv
                sig = 1.0 / (1.0 + jnp.exp(-accs))
                out_v[pl.ds(off, LANES)] = sig * (HI - LO) + LO

            @pl.when(any_tail)
            def _():
                dot(True)

            @pl.when(jnp.logical_not(any_tail))
            def _():
                dot(False)

            return carry

        lax.fori_loop(0, chunks, chunk_body, 0)
        pltpu.sync_copy(out_v, out_hbm.at[pl.ds(base, b_per_w)])

    return sc_kernel


def kernel(user_idx, movie_idx, user_emb_table, item_emb_table,
           user_bias_table, item_bias_table):
    batch = user_idx.shape[0]
    n_rows, emb = user_emb_table.shape
    n_main = (n_rows // TILE_MINOR) * TILE_MINOR
    tq = n_main // TILE_MINOR
    te = emb // TILE_MAJOR

    def flat_view(t):
        return (t[:n_main].T
                .reshape(te, TILE_MAJOR, tq, TILE_MINOR)
                .transpose(0, 2, 1, 3)
                .reshape(-1))

    sc = _build_sc_kernel(batch, emb, n_main, n_rows)
    return sc(
        user_idx.astype(jnp.int32),
        movie_idx.astype(jnp.int32),
        flat_view(user_emb_table),
        flat_view(item_emb_table),
        user_bias_table.T,
        item_bias_table.T,
        user_emb_table[n_main:].T,
        item_emb_table[n_main:].T,
        user_bias_table[n_main:].reshape(-1),
        item_bias_table[n_main:].reshape(-1),
    )
